# Initial kernel scaffold; baseline (speedup 1.0000x reference)
import jax
import jax.numpy as jnp
from jax.experimental import pallas as pl


def kernel(input_spec):
    raise NotImplementedError("write your pallas kernel here")


# speedup vs baseline: 5.0752x; 5.0752x over previous
import functools

import jax
import jax.numpy as jnp
import numpy as np
from jax import lax
from jax.experimental import pallas as pl
from jax.experimental.pallas import tpu as pltpu
from jax.experimental.pallas import tpu_sc as plsc

_MASK_RATIO = 0.2
_MASK_KEY = 42

_LANES = 16
_NC, _NS = 2, 16


@functools.lru_cache(maxsize=None)
def _mask_indices(batch: int, time: int) -> np.ndarray:
    num_masks = int(_MASK_RATIO * time)
    with jax.ensure_compile_time_eval():
        keys = jax.random.split(jax.random.key(_MASK_KEY), batch)
        idx = jax.vmap(
            lambda k: jax.random.choice(k, time, shape=(num_masks,), replace=False)
        )(keys)
        idx = np.asarray(jax.device_get(idx), dtype=np.int32)
    pad = (-num_masks) % _LANES
    if pad:
        idx = np.concatenate([idx, np.repeat(idx[:, -1:], pad, axis=1)], axis=1)
    return idx


def _build_mask_sc(idx: jax.Array, batch: int, t_pad: int) -> jax.Array:
    n_pad = idx.shape[1]
    mesh = plsc.VectorSubcoreMesh(core_axis_name="c", subcore_axis_name="s")

    @functools.partial(
        pl.kernel,
        out_type=jax.ShapeDtypeStruct((batch, t_pad), jnp.float32),
        mesh=mesh,
        compiler_params=pltpu.CompilerParams(needs_layout_passes=False),
        scratch_types=[
            pltpu.VMEM((n_pad,), jnp.int32),
            pltpu.VMEM((t_pad,), jnp.float32),
        ],
    )
    def sc_kernel(idx_hbm, mask_hbm, idx_v, row_v):
        wid = lax.axis_index("s") * _NC + lax.axis_index("c")

        ones16 = jnp.ones((_LANES,), jnp.float32)
        zeros16 = jnp.zeros((_LANES,), jnp.float32)

        def fill(i, carry):
            row_v[pl.ds(i * _LANES, _LANES)] = ones16
            return carry

        lax.fori_loop(0, t_pad // _LANES, fill, 0)

        pltpu.sync_copy(idx_hbm.at[wid], idx_v)

        def scat(i, carry):
            iv = idx_v[pl.ds(i * _LANES, _LANES)]
            plsc.store_scatter(row_v, [iv], zeros16)
            return carry

        lax.fori_loop(0, n_pad // _LANES, scat, 0)

        pltpu.sync_copy(row_v, mask_hbm.at[wid])

    return sc_kernel(idx)


def _apply_mask_tc(x: jax.Array, mask: jax.Array) -> jax.Array:
    batch, freq, time = x.shape
    t_pad = mask.shape[-1]
    mask3 = mask.reshape(batch, 1, t_pad)

    def body(x_ref, m_ref, o_ref):
        o_ref[...] = x_ref[...] * m_ref[:, :, :time]

    return pl.pallas_call(
        body,
        grid=(batch,),
        in_specs=[
            pl.BlockSpec((1, freq, time), lambda b: (b, 0, 0)),
            pl.BlockSpec((1, 1, t_pad), lambda b: (b, 0, 0)),
        ],
        out_specs=pl.BlockSpec((1, freq, time), lambda b: (b, 0, 0)),
        out_shape=jax.ShapeDtypeStruct((batch, freq, time), jnp.float32),
    )(x, mask3)


def kernel(input_spec):
    batch, freq, time = input_spec.shape
    t_pad = time + ((-time) % _LANES)
    idx = jnp.asarray(_mask_indices(batch, time))
    mask = _build_mask_sc(idx, batch, t_pad)
    return _apply_mask_tc(input_spec, mask)


# speedup vs baseline: 6.4259x; 1.2661x over previous
import functools

import jax
import jax.numpy as jnp
import numpy as np
from jax import lax
from jax.experimental import pallas as pl
from jax.experimental.pallas import tpu as pltpu
from jax.experimental.pallas import tpu_sc as plsc

_MASK_RATIO = 0.2
_MASK_KEY = 42

_LANES = 16
_NC, _NS = 2, 16


@functools.lru_cache(maxsize=None)
def _mask_indices(batch: int, time: int) -> np.ndarray:
    num_masks = int(_MASK_RATIO * time)
    with jax.ensure_compile_time_eval():
        keys = jax.random.split(jax.random.key(_MASK_KEY), batch)
        idx = jax.vmap(
            lambda k: jax.random.choice(k, time, shape=(num_masks,), replace=False)
        )(keys)
        idx = np.asarray(jax.device_get(idx), dtype=np.int32)
    pad = (-num_masks) % _LANES
    if pad:
        idx = np.concatenate([idx, np.repeat(idx[:, -1:], pad, axis=1)], axis=1)
    return idx


def _build_mask_sc(idx: jax.Array, batch: int, t_pad: int) -> jax.Array:
    n_pad = idx.shape[1]
    mesh = plsc.VectorSubcoreMesh(core_axis_name="c", subcore_axis_name="s")

    @functools.partial(
        pl.kernel,
        out_type=jax.ShapeDtypeStruct((batch, t_pad), jnp.float32),
        mesh=mesh,
        compiler_params=pltpu.CompilerParams(needs_layout_passes=False),
        scratch_types=[
            pltpu.VMEM((n_pad,), jnp.int32),
            pltpu.VMEM((t_pad,), jnp.float32),
        ],
    )
    def sc_kernel(idx_hbm, mask_hbm, idx_v, row_v):
        wid = lax.axis_index("s") * _NC + lax.axis_index("c")

        ones16 = jnp.ones((_LANES,), jnp.float32)
        zeros16 = jnp.zeros((_LANES,), jnp.float32)

        def fill(i, carry):
            row_v[pl.ds(i * _LANES, _LANES)] = ones16
            return carry

        lax.fori_loop(0, t_pad // _LANES, fill, 0)

        pltpu.sync_copy(idx_hbm.at[wid], idx_v)

        def scat(i, carry):
            iv = idx_v[pl.ds(i * _LANES, _LANES)]
            plsc.store_scatter(row_v, [iv], zeros16)
            return carry

        lax.fori_loop(0, n_pad // _LANES, scat, 0)

        pltpu.sync_copy(row_v, mask_hbm.at[wid])

    return sc_kernel(idx)


def _apply_mask_tc(x: jax.Array, mask: jax.Array) -> jax.Array:
    batch, freq, time = x.shape
    t_pad = mask.shape[-1]
    mask3 = mask.reshape(batch, 1, t_pad)

    bb = 4

    def body(x_ref, m_ref, o_ref):
        o_ref[...] = x_ref[...] * m_ref[:, :, :time]

    return pl.pallas_call(
        body,
        grid=(batch // bb,),
        in_specs=[
            pl.BlockSpec((bb, freq, time), lambda b: (b, 0, 0)),
            pl.BlockSpec((bb, 1, t_pad), lambda b: (b, 0, 0)),
        ],
        out_specs=pl.BlockSpec((bb, freq, time), lambda b: (b, 0, 0)),
        out_shape=jax.ShapeDtypeStruct((batch, freq, time), jnp.float32),
        compiler_params=pltpu.CompilerParams(
            vmem_limit_bytes=100 * 1024 * 1024,
        ),
    )(x, mask3)


def kernel(input_spec):
    batch, freq, time = input_spec.shape
    t_pad = time + ((-time) % _LANES)
    idx = jnp.asarray(_mask_indices(batch, time))
    mask = _build_mask_sc(idx, batch, t_pad)
    return _apply_mask_tc(input_spec, mask)


# speedup vs baseline: 6.6154x; 1.0295x over previous
import functools

import jax
import jax.numpy as jnp
import numpy as np
from jax import lax
from jax.experimental import pallas as pl
from jax.experimental.pallas import tpu as pltpu
from jax.experimental.pallas import tpu_sc as plsc

_MASK_RATIO = 0.2
_MASK_KEY = 42

_LANES = 16
_NC, _NS = 2, 16


@functools.lru_cache(maxsize=None)
def _mask_indices(batch: int, time: int) -> np.ndarray:
    num_masks = int(_MASK_RATIO * time)
    with jax.ensure_compile_time_eval():
        keys = jax.random.split(jax.random.key(_MASK_KEY), batch)
        idx = jax.vmap(
            lambda k: jax.random.choice(k, time, shape=(num_masks,), replace=False)
        )(keys)
        idx = np.asarray(jax.device_get(idx), dtype=np.int32)
    pad = (-num_masks) % _LANES
    if pad:
        idx = np.concatenate([idx, np.repeat(idx[:, -1:], pad, axis=1)], axis=1)
    return idx


def _build_mask_sc(idx: jax.Array, batch: int, t_pad: int) -> jax.Array:
    n_pad = idx.shape[1]
    mesh = plsc.VectorSubcoreMesh(core_axis_name="c", subcore_axis_name="s")

    @functools.partial(
        pl.kernel,
        out_type=jax.ShapeDtypeStruct((batch, t_pad), jnp.float32),
        mesh=mesh,
        compiler_params=pltpu.CompilerParams(needs_layout_passes=False),
        scratch_types=[
            pltpu.VMEM((n_pad,), jnp.int32),
            pltpu.VMEM((t_pad,), jnp.float32),
        ],
    )
    def sc_kernel(idx_hbm, mask_hbm, idx_v, row_v):
        wid = lax.axis_index("s") * _NC + lax.axis_index("c")

        ones16 = jnp.ones((_LANES,), jnp.float32)
        zeros16 = jnp.zeros((_LANES,), jnp.float32)

        def fill(i, carry):
            row_v[pl.ds(i * _LANES, _LANES)] = ones16
            return carry

        lax.fori_loop(0, t_pad // _LANES, fill, 0)

        pltpu.sync_copy(idx_hbm.at[wid], idx_v)

        def scat(i, carry):
            iv = idx_v[pl.ds(i * _LANES, _LANES)]
            plsc.store_scatter(row_v, [iv], zeros16)
            return carry

        lax.fori_loop(0, n_pad // _LANES, scat, 0)

        pltpu.sync_copy(row_v, mask_hbm.at[wid])

    return sc_kernel(idx)


def _apply_mask_tc(x: jax.Array, mask: jax.Array) -> jax.Array:
    batch, freq, time = x.shape
    t_pad = mask.shape[-1]
    mask3 = mask.reshape(batch, 1, t_pad)

    bb = 8

    def body(x_ref, m_ref, o_ref):
        o_ref[...] = x_ref[...] * m_ref[:, :, :time]

    return pl.pallas_call(
        body,
        grid=(batch // bb,),
        in_specs=[
            pl.BlockSpec((bb, freq, time), lambda b: (b, 0, 0)),
            pl.BlockSpec((bb, 1, t_pad), lambda b: (b, 0, 0)),
        ],
        out_specs=pl.BlockSpec((bb, freq, time), lambda b: (b, 0, 0)),
        out_shape=jax.ShapeDtypeStruct((batch, freq, time), jnp.float32),
        compiler_params=pltpu.CompilerParams(
            vmem_limit_bytes=100 * 1024 * 1024,
        ),
    )(x, mask3)


def kernel(input_spec):
    batch, freq, time = input_spec.shape
    t_pad = time + ((-time) % _LANES)
    idx = jnp.asarray(_mask_indices(batch, time))
    mask = _build_mask_sc(idx, batch, t_pad)
    return _apply_mask_tc(input_spec, mask)


# speedup vs baseline: 6.7138x; 1.0149x over previous
import functools

import jax
import jax.numpy as jnp
import numpy as np
from jax import lax
from jax.experimental import pallas as pl
from jax.experimental.pallas import tpu as pltpu
from jax.experimental.pallas import tpu_sc as plsc

_MASK_RATIO = 0.2
_MASK_KEY = 42

_LANES = 16
_NC, _NS = 2, 16


@functools.lru_cache(maxsize=None)
def _mask_indices(batch: int, time: int) -> np.ndarray:
    num_masks = int(_MASK_RATIO * time)
    with jax.ensure_compile_time_eval():
        keys = jax.random.split(jax.random.key(_MASK_KEY), batch)
        idx = jax.vmap(
            lambda k: jax.random.choice(k, time, shape=(num_masks,), replace=False)
        )(keys)
        idx = np.asarray(jax.device_get(idx), dtype=np.int32)
    pad = (-num_masks) % _LANES
    if pad:
        idx = np.concatenate([idx, np.repeat(idx[:, -1:], pad, axis=1)], axis=1)
    return idx


def _build_mask_sc(idx: jax.Array, batch: int, t_pad: int) -> jax.Array:
    n_pad = idx.shape[1]
    mesh = plsc.VectorSubcoreMesh(core_axis_name="c", subcore_axis_name="s")

    @functools.partial(
        pl.kernel,
        out_type=jax.ShapeDtypeStruct((batch, t_pad), jnp.float32),
        mesh=mesh,
        compiler_params=pltpu.CompilerParams(needs_layout_passes=False),
        scratch_types=[
            pltpu.VMEM((n_pad,), jnp.int32),
            pltpu.VMEM((t_pad,), jnp.float32),
        ],
    )
    def sc_kernel(idx_hbm, mask_hbm, idx_v, row_v):
        wid = lax.axis_index("s") * _NC + lax.axis_index("c")

        ones16 = jnp.ones((_LANES,), jnp.float32)
        zeros16 = jnp.zeros((_LANES,), jnp.float32)

        def fill(i, carry):
            row_v[pl.ds(i * _LANES, _LANES)] = ones16
            return carry

        lax.fori_loop(0, t_pad // _LANES, fill, 0)

        pltpu.sync_copy(idx_hbm.at[wid], idx_v)

        def scat(i, carry):
            iv = idx_v[pl.ds(i * _LANES, _LANES)]
            plsc.store_scatter(row_v, [iv], zeros16)
            return carry

        lax.fori_loop(0, n_pad // _LANES, scat, 0)

        pltpu.sync_copy(row_v, mask_hbm.at[wid])

    return sc_kernel(idx)


def _apply_mask_tc(x: jax.Array, mask: jax.Array) -> jax.Array:
    batch, freq, time = x.shape
    t_pad = mask.shape[-1]
    mask3 = mask.reshape(batch, 1, t_pad)

    bb = 16

    def body(x_ref, m_ref, o_ref):
        o_ref[...] = x_ref[...] * m_ref[:, :, :time]

    return pl.pallas_call(
        body,
        grid=(batch // bb,),
        in_specs=[
            pl.BlockSpec((bb, freq, time), lambda b: (b, 0, 0)),
            pl.BlockSpec((bb, 1, t_pad), lambda b: (b, 0, 0)),
        ],
        out_specs=pl.BlockSpec((bb, freq, time), lambda b: (b, 0, 0)),
        out_shape=jax.ShapeDtypeStruct((batch, freq, time), jnp.float32),
        compiler_params=pltpu.CompilerParams(
            vmem_limit_bytes=100 * 1024 * 1024,
        ),
    )(x, mask3)


def kernel(input_spec):
    batch, freq, time = input_spec.shape
    t_pad = time + ((-time) % _LANES)
    idx = jnp.asarray(_mask_indices(batch, time))
    mask = _build_mask_sc(idx, batch, t_pad)
    return _apply_mask_tc(input_spec, mask)
